# Initial kernel scaffold; baseline (speedup 1.0000x reference)
import jax
import jax.numpy as jnp
from jax.experimental import pallas as pl


def kernel(X, edge_index, W1, b1, W2, b2, W3, b3):
    raise NotImplementedError("write your pallas kernel here")


# speedup vs baseline: 10.0926x; 10.0926x over previous
import functools

import jax
import jax.numpy as jnp
from jax import lax
from jax.experimental import pallas as pl
from jax.experimental.pallas import tpu as pltpu
from jax.experimental.pallas import tpu_sc as plsc

N = 10000
D = 128
E = 320000

NC = 2
NS = 16
NW = NC * NS
C = 128
CHUNKS = (E + NW * C - 1) // (NW * C)
E_PAD = NW * CHUNKS * C
N_PAD = 10240
ROWS_PER_TILE = N_PAD // NS

_mesh = plsc.VectorSubcoreMesh(core_axis_name="c", subcore_axis_name="s")


@functools.partial(
    pl.kernel,
    out_type=jax.ShapeDtypeStruct((NC, N_PAD, D), jnp.float32),
    mesh=_mesh,
    scratch_types=[
        pltpu.VMEM((CHUNKS, C), jnp.int32),
        pltpu.VMEM((C, D), jnp.float32),
        pltpu.VMEM_SHARED((N_PAD, D), jnp.float32),
    ],
)
def _deg_kernel(dst_hbm, ones_hbm, zeros_hbm, out_hbm, dst2d, ones_v, acc):
    cid = lax.axis_index("c")
    sid = lax.axis_index("s")
    w = cid * NS + sid
    pltpu.sync_copy(dst_hbm.at[w], dst2d)
    pltpu.sync_copy(ones_hbm, ones_v)
    pltpu.sync_copy(
        zeros_hbm.at[pl.ds(sid * ROWS_PER_TILE, ROWS_PER_TILE)],
        acc.at[pl.ds(sid * ROWS_PER_TILE, ROWS_PER_TILE)],
    )
    plsc.subcore_barrier()

    def body(j, carry):
        pltpu.sync_copy(ones_v, acc.at[dst2d.at[j]], add=True)
        return carry

    lax.fori_loop(0, CHUNKS, body, 0)
    plsc.subcore_barrier()
    pltpu.sync_copy(
        acc.at[pl.ds(sid * ROWS_PER_TILE, ROWS_PER_TILE)],
        out_hbm.at[cid, pl.ds(sid * ROWS_PER_TILE, ROWS_PER_TILE)],
    )


@functools.partial(
    pl.kernel,
    out_type=jax.ShapeDtypeStruct((NC, N_PAD, D), jnp.float32),
    mesh=_mesh,
    scratch_types=[
        pltpu.VMEM((CHUNKS, C), jnp.int32),
        pltpu.VMEM((CHUNKS, C), jnp.int32),
        pltpu.VMEM((C, D), jnp.float32),
        pltpu.VMEM_SHARED((N_PAD, D), jnp.float32),
        pltpu.SemaphoreType.DMA,
    ],
)
def _scatter_kernel(g_hbm, src_hbm, dst_hbm, zeros_hbm, out_hbm,
                    src2d, dst2d, rows, acc, sem):
    cid = lax.axis_index("c")
    sid = lax.axis_index("s")
    w = cid * NS + sid
    pltpu.sync_copy(src_hbm.at[w], src2d)
    pltpu.sync_copy(dst_hbm.at[w], dst2d)
    pltpu.sync_copy(
        zeros_hbm.at[pl.ds(sid * ROWS_PER_TILE, ROWS_PER_TILE)],
        acc.at[pl.ds(sid * ROWS_PER_TILE, ROWS_PER_TILE)],
    )
    plsc.subcore_barrier()

    def body(j, carry):
        pltpu.async_copy(g_hbm.at[src2d.at[j]], rows, sem).wait()
        pltpu.sync_copy(rows, acc.at[dst2d.at[j]], add=True)
        return carry

    lax.fori_loop(0, CHUNKS, body, 0)
    plsc.subcore_barrier()
    pltpu.sync_copy(
        acc.at[pl.ds(sid * ROWS_PER_TILE, ROWS_PER_TILE)],
        out_hbm.at[cid, pl.ds(sid * ROWS_PER_TILE, ROWS_PER_TILE)],
    )


def _dis_block(dp_ref):
    deg = dp_ref[0, :, 0:1] + dp_ref[1, :, 0:1] + 1.0
    return lax.rsqrt(deg)


def _first_body(dp_ref, x_ref, w_ref, o_ref):
    dis = _dis_block(dp_ref)
    o_ref[...] = dis * jnp.dot(x_ref[...], w_ref[...],
                               preferred_element_type=jnp.float32)


def _mid_body(dp_ref, t_ref, g_ref, b_ref, w_ref, o_ref):
    dis = _dis_block(dp_ref)
    u = dis * (t_ref[0] + t_ref[1] + g_ref[...]) + b_ref[...]
    x = jnp.maximum(u, 0.0)
    o_ref[...] = dis * jnp.dot(x, w_ref[...],
                               preferred_element_type=jnp.float32)


def _last_body(dp_ref, t_ref, g_ref, b_ref, o_ref):
    dis = _dis_block(dp_ref)
    o_ref[...] = dis * (t_ref[0] + t_ref[1] + g_ref[...]) + b_ref[...]


_BLK = 512
_GRID = N_PAD // _BLK

_dp_spec = pl.BlockSpec((NC, _BLK, D), lambda i: (0, i, 0))
_t_spec = pl.BlockSpec((NC, _BLK, D), lambda i: (0, i, 0))
_row_spec = pl.BlockSpec((_BLK, D), lambda i: (i, 0))
_w_spec = pl.BlockSpec((D, D), lambda i: (0, 0))
_b_spec = pl.BlockSpec((1, D), lambda i: (0, 0))
_out_shape = jax.ShapeDtypeStruct((N_PAD, D), jnp.float32)

_first_tc = pl.pallas_call(
    _first_body, grid=(_GRID,),
    in_specs=[_dp_spec, _row_spec, _w_spec],
    out_specs=_row_spec, out_shape=_out_shape)

_mid_tc = pl.pallas_call(
    _mid_body, grid=(_GRID,),
    in_specs=[_dp_spec, _t_spec, _row_spec, _b_spec, _w_spec],
    out_specs=_row_spec, out_shape=_out_shape)

_last_tc = pl.pallas_call(
    _last_body, grid=(_GRID,),
    in_specs=[_dp_spec, _t_spec, _row_spec, _b_spec],
    out_specs=_row_spec, out_shape=_out_shape)


def kernel(X, edge_index, W1, b1, W2, b2, W3, b3):
    src = edge_index[0].astype(jnp.int32)
    dst = edge_index[1].astype(jnp.int32)
    pad = jnp.full((E_PAD - E,), N, dtype=jnp.int32)
    src3 = jnp.reshape(jnp.concatenate([src, pad]), (NW, CHUNKS, C))
    dst3 = jnp.reshape(jnp.concatenate([dst, pad]), (NW, CHUNKS, C))

    x_pad = jnp.zeros((N_PAD, D), jnp.float32).at[:N].set(X)
    onesD = jnp.ones((C, D), jnp.float32)
    zerosD = jnp.zeros((N_PAD, D), jnp.float32)

    dp = _deg_kernel(dst3, onesD, zerosD)

    b1r = jnp.reshape(b1, (1, D))
    b2r = jnp.reshape(b2, (1, D))
    b3r = jnp.reshape(b3, (1, D))

    g1 = _first_tc(dp, x_pad, W1)
    t1 = _scatter_kernel(g1, src3, dst3, zerosD)
    g2 = _mid_tc(dp, t1, g1, b1r, W2)
    t2 = _scatter_kernel(g2, src3, dst3, zerosD)
    g3 = _mid_tc(dp, t2, g2, b2r, W3)
    t3 = _scatter_kernel(g3, src3, dst3, zerosD)
    out = _last_tc(dp, t3, g3, b3r)
    return out[:N]


# speedup vs baseline: 10.9315x; 1.0831x over previous
import functools

import jax
import jax.numpy as jnp
from jax import lax
from jax.experimental import pallas as pl
from jax.experimental.pallas import tpu as pltpu
from jax.experimental.pallas import tpu_sc as plsc

N = 10000
D = 128
DH = D // 2
E = 320000

NC = 2
NS = 16
NW = NC * NS
C = 128
NBUF = 4
DCHUNKS = -(-(E // NW) // C) * C // C
DCHUNKS = -(-(E // NW) // (C * NBUF)) * NBUF
E_PAD = NW * DCHUNKS * C
SCHUNKS = E_PAD // (NS * C)
SGROUPS = SCHUNKS // NBUF
N_PAD = 10240
RPT = N_PAD // NS

_mesh = plsc.VectorSubcoreMesh(core_axis_name="c", subcore_axis_name="s")
_sc_params = pltpu.CompilerParams(use_tc_tiling_on_sc=False)


@functools.partial(
    pl.kernel,
    out_type=jax.ShapeDtypeStruct((NC, N_PAD, DH), jnp.float32),
    mesh=_mesh,
    compiler_params=_sc_params,
    scratch_types=[
        pltpu.VMEM((DCHUNKS, C), jnp.int32),
        pltpu.VMEM((C, DH), jnp.float32),
        pltpu.VMEM_SHARED((N_PAD, DH), jnp.float32),
    ],
)
def _deg_kernel(dst_hbm, ones_hbm, zeros_hbm, out_hbm, dst2d, ones_v, acc):
    cid = lax.axis_index("c")
    sid = lax.axis_index("s")
    w = cid * NS + sid
    pltpu.sync_copy(dst_hbm.at[w], dst2d)
    pltpu.sync_copy(ones_hbm, ones_v)
    pltpu.sync_copy(zeros_hbm.at[pl.ds(sid * RPT, RPT)],
                    acc.at[pl.ds(sid * RPT, RPT)])
    plsc.subcore_barrier()

    def body(j, carry):
        pltpu.sync_copy(ones_v, acc.at[dst2d.at[j]], add=True)
        return carry

    lax.fori_loop(0, DCHUNKS, body, 0)
    plsc.subcore_barrier()
    pltpu.sync_copy(acc.at[pl.ds(sid * RPT, RPT)],
                    out_hbm.at[cid, pl.ds(sid * RPT, RPT)])


@functools.partial(
    pl.kernel,
    out_type=jax.ShapeDtypeStruct((NC, N_PAD, DH), jnp.float32),
    mesh=_mesh,
    compiler_params=_sc_params,
    scratch_types=[
        pltpu.VMEM((SCHUNKS, C), jnp.int32),
        pltpu.VMEM((SCHUNKS, C), jnp.int32),
        pltpu.VMEM((C, DH), jnp.float32),
        pltpu.VMEM((C, DH), jnp.float32),
        pltpu.VMEM((C, DH), jnp.float32),
        pltpu.VMEM((C, DH), jnp.float32),
        pltpu.VMEM_SHARED((N_PAD, DH), jnp.float32),
        pltpu.SemaphoreType.DMA,
        pltpu.SemaphoreType.DMA,
        pltpu.SemaphoreType.DMA,
        pltpu.SemaphoreType.DMA,
    ],
)
def _scatter_kernel(g_hbm, src_hbm, dst_hbm, zeros_hbm, out_hbm,
                    src2d, dst2d, rows0, rows1, rows2, rows3, acc,
                    sem0, sem1, sem2, sem3):
    rows = (rows0, rows1, rows2, rows3)
    sems = (sem0, sem1, sem2, sem3)
    cid = lax.axis_index("c")
    sid = lax.axis_index("s")
    g_half = g_hbm.at[cid]
    pltpu.sync_copy(src_hbm.at[sid], src2d)
    pltpu.sync_copy(dst_hbm.at[sid], dst2d)
    pltpu.sync_copy(zeros_hbm.at[pl.ds(sid * RPT, RPT)],
                    acc.at[pl.ds(sid * RPT, RPT)])
    plsc.subcore_barrier()

    for b in range(NBUF):
        pltpu.async_copy(g_half.at[src2d.at[b]], rows[b], sems[b])

    def body(gidx, carry):
        for b in range(NBUF):
            j = gidx * NBUF + b
            pltpu.make_async_copy(g_half.at[src2d.at[j]], rows[b],
                                  sems[b]).wait()
            pltpu.sync_copy(rows[b], acc.at[dst2d.at[j]], add=True)
            pltpu.async_copy(g_half.at[src2d.at[j + NBUF]], rows[b], sems[b])
        return carry

    lax.fori_loop(0, SGROUPS - 1, body, 0)
    for b in range(NBUF):
        j = (SGROUPS - 1) * NBUF + b
        pltpu.make_async_copy(g_half.at[src2d.at[j]], rows[b], sems[b]).wait()
        pltpu.sync_copy(rows[b], acc.at[dst2d.at[j]], add=True)
    plsc.subcore_barrier()
    pltpu.sync_copy(acc.at[pl.ds(sid * RPT, RPT)],
                    out_hbm.at[cid, pl.ds(sid * RPT, RPT)])


def _dis_block(dp_ref):
    deg = dp_ref[0, :, 0:1] + dp_ref[1, :, 0:1] + 1.0
    return lax.rsqrt(deg)


def _split_store(o_ref, res):
    o_ref[0] = res[:, :DH]
    o_ref[1] = res[:, DH:]


def _first_body(dp_ref, x_ref, w_ref, o_ref):
    dis = _dis_block(dp_ref)
    _split_store(o_ref, dis * jnp.dot(x_ref[...], w_ref[...],
                                      preferred_element_type=jnp.float32))


def _mid_body(dp_ref, t_ref, g_ref, b_ref, w_ref, o_ref):
    dis = _dis_block(dp_ref)
    tg = jnp.concatenate([t_ref[0] + g_ref[0], t_ref[1] + g_ref[1]], axis=1)
    x = jnp.maximum(dis * tg + b_ref[...], 0.0)
    _split_store(o_ref, dis * jnp.dot(x, w_ref[...],
                                      preferred_element_type=jnp.float32))


def _last_body(dp_ref, t_ref, g_ref, b_ref, o_ref):
    dis = _dis_block(dp_ref)
    tg = jnp.concatenate([t_ref[0] + g_ref[0], t_ref[1] + g_ref[1]], axis=1)
    o_ref[...] = dis * tg + b_ref[...]


_BLK = 512
_GRID = N_PAD // _BLK

_h_spec = pl.BlockSpec((NC, _BLK, DH), lambda i: (0, i, 0))
_row_spec = pl.BlockSpec((_BLK, D), lambda i: (i, 0))
_w_spec = pl.BlockSpec((D, D), lambda i: (0, 0))
_b_spec = pl.BlockSpec((1, D), lambda i: (0, 0))
_split_shape = jax.ShapeDtypeStruct((NC, N_PAD, DH), jnp.float32)

_first_tc = pl.pallas_call(
    _first_body, grid=(_GRID,),
    in_specs=[_h_spec, _row_spec, _w_spec],
    out_specs=_h_spec, out_shape=_split_shape)

_mid_tc = pl.pallas_call(
    _mid_body, grid=(_GRID,),
    in_specs=[_h_spec, _h_spec, _h_spec, _b_spec, _w_spec],
    out_specs=_h_spec, out_shape=_split_shape)

_last_tc = pl.pallas_call(
    _last_body, grid=(_GRID,),
    in_specs=[_h_spec, _h_spec, _h_spec, _b_spec],
    out_specs=_row_spec,
    out_shape=jax.ShapeDtypeStruct((N_PAD, D), jnp.float32))


def kernel(X, edge_index, W1, b1, W2, b2, W3, b3):
    src = edge_index[0].astype(jnp.int32)
    dst = edge_index[1].astype(jnp.int32)
    pad = jnp.full((E_PAD - E,), N, dtype=jnp.int32)
    src_flat = jnp.concatenate([src, pad])
    dst_flat = jnp.concatenate([dst, pad])
    dst3deg = jnp.reshape(dst_flat, (NW, DCHUNKS, C))
    src3 = jnp.reshape(src_flat, (NS, SCHUNKS, C))
    dst3 = jnp.reshape(dst_flat, (NS, SCHUNKS, C))

    x_pad = jnp.zeros((N_PAD, D), jnp.float32).at[:N].set(X)
    onesH = jnp.ones((C, DH), jnp.float32)
    zerosH = jnp.zeros((N_PAD, DH), jnp.float32)

    dp = _deg_kernel(dst3deg, onesH, zerosH)

    b1r = jnp.reshape(b1, (1, D))
    b2r = jnp.reshape(b2, (1, D))
    b3r = jnp.reshape(b3, (1, D))

    g1 = _first_tc(dp, x_pad, W1)
    t1 = _scatter_kernel(g1, src3, dst3, zerosH)
    g2 = _mid_tc(dp, t1, g1, b1r, W2)
    t2 = _scatter_kernel(g2, src3, dst3, zerosH)
    g3 = _mid_tc(dp, t2, g2, b2r, W3)
    t3 = _scatter_kernel(g3, src3, dst3, zerosH)
    out = _last_tc(dp, t3, g3, b3r)
    return out[:N]
